# Initial kernel scaffold; baseline (speedup 1.0000x reference)
import jax
import jax.numpy as jnp
from jax.experimental import pallas as pl


def kernel(x, edge_index, batch, W, b, w_pool):
    raise NotImplementedError("write your pallas kernel here")


# speedup vs baseline: 5.1461x; 5.1461x over previous
import functools

import jax
import jax.numpy as jnp
from jax.experimental import pallas as pl

N = 10000
D = 128
C = 16
K = 5000


def _body(x_ref, w_ref, b_ref, wp_ref, xf_ref, pred_ref):
    x = x_ref[:, :]
    w = w_ref[:, :]
    b = b_ref[:, :]
    wp = wp_ref[:, :]

    pred = jax.lax.dot_general(
        x, w, (((1,), (1,)), ((), ())), preferred_element_type=jnp.float32
    ) + b
    pred_ref[:, :] = pred

    z = jax.lax.dot_general(
        wp, pred, (((1,), (1,)), ((), ())), preferred_element_type=jnp.float32
    )
    norm = jnp.sqrt(jnp.sum(wp * wp)) + 1e-16
    s = jnp.tanh(z / norm)

    u = jax.lax.bitcast_convert_type(z, jnp.uint32)
    sign = u >> jnp.uint32(31)
    flip = jnp.where(sign == jnp.uint32(1),
                     jnp.uint32(0xFFFFFFFF), jnp.uint32(0x80000000))
    key = u ^ flip

    def _cnt_ge(t):
        return jnp.sum((key >= t).astype(jnp.int32))

    def _bis(_, lh):
        lo, hi = lh
        mid = hi - (hi - lo) // jnp.uint32(2)
        ge = _cnt_ge(mid) >= K
        return (jnp.where(ge, mid, lo), jnp.where(ge, hi, mid - jnp.uint32(1)))

    kth, _ = jax.lax.fori_loop(
        0, 32, _bis, (jnp.uint32(0), jnp.uint32(0xFFFFFFFF)))

    above = key > kth
    m = jnp.sum(above.astype(jnp.int32))
    need = K - m

    tie = key == kth
    idx = jax.lax.broadcasted_iota(jnp.int32, (1, N), 1)

    def _bis_idx(_, lh):
        lo, hi = lh
        mid = (lo + hi) // 2
        c = jnp.sum((tie & (idx <= mid)).astype(jnp.int32))
        ok = c >= need
        return (jnp.where(ok, lo, mid + 1), jnp.where(ok, mid, hi))

    _, jstar = jax.lax.fori_loop(0, 14, _bis_idx, (0, N - 1))

    sel = above | (tie & (idx <= jstar))
    wgt = jnp.where(sel, s, 0.0)

    acc = jax.lax.dot_general(
        wgt, pred, (((1,), (0,)), ((), ())), preferred_element_type=jnp.float32
    )
    xf_ref[:, :] = acc * (1.0 / K)


@functools.partial(jax.jit, static_argnames=())
def kernel(x, edge_index, batch, W, b, w_pool):
    del edge_index, batch
    b2 = b.reshape(1, C)
    wp2 = w_pool.reshape(1, C)
    x_final, pred = pl.pallas_call(
        _body,
        out_shape=(
            jax.ShapeDtypeStruct((1, C), jnp.float32),
            jax.ShapeDtypeStruct((N, C), jnp.float32),
        ),
    )(x, W, b2, wp2)
    return (x_final, pred)


# speedup vs baseline: 5.7704x; 1.1213x over previous
import functools

import jax
import jax.numpy as jnp
from jax.experimental import pallas as pl
from jax.experimental.pallas import tpu as pltpu

N = 10000
D = 128
C = 16
K = 5000
R = 10
M = N // R


def _body(x_ref, w_ref, b_ref, wp_ref, xf_ref, pred_ref, zs_ref):
    x = x_ref[:, :]
    w = w_ref[:, :]
    b = b_ref[:, :]
    wp = wp_ref[:, :]

    pred = jax.lax.dot_general(
        x, w, (((1,), (1,)), ((), ())), preferred_element_type=jnp.float32
    ) + b
    pred_ref[:, :] = pred

    for j in range(R):
        zj = jax.lax.dot_general(
            wp, pred[j * M:(j + 1) * M, :], (((1,), (1,)), ((), ())),
            preferred_element_type=jnp.float32)
        zs_ref[j:j + 1, :] = zj
    z = zs_ref[:, :]

    u = jax.lax.bitcast_convert_type(z, jnp.uint32)
    sign = u >> jnp.uint32(31)
    flip = jnp.where(sign == jnp.uint32(1),
                     jnp.uint32(0xFFFFFFFF), jnp.uint32(0x80000000))
    key = u ^ flip

    def _cnt_ge(t):
        return jnp.sum((key >= t).astype(jnp.int32))

    def _bis(_, lh):
        lo, hi = lh
        mid = hi - (hi - lo) // jnp.uint32(2)
        ge = _cnt_ge(mid) >= K
        return (jnp.where(ge, mid, lo), jnp.where(ge, hi, mid - jnp.uint32(1)))

    kth, _ = jax.lax.fori_loop(
        0, 32, _bis, (jnp.uint32(0), jnp.uint32(0xFFFFFFFF)))

    above = key > kth
    m = jnp.sum(above.astype(jnp.int32))
    need = K - m

    tie = key == kth
    idx = (jax.lax.broadcasted_iota(jnp.int32, (R, M), 0) * M
           + jax.lax.broadcasted_iota(jnp.int32, (R, M), 1))

    def _bis_idx(_, lh):
        lo, hi = lh
        mid = (lo + hi) // 2
        c = jnp.sum((tie & (idx <= mid)).astype(jnp.int32))
        ok = c >= need
        return (jnp.where(ok, lo, mid + 1), jnp.where(ok, mid, hi))

    _, jstar = jax.lax.fori_loop(0, 14, _bis_idx, (0, N - 1))

    sel = above | (tie & (idx <= jstar))
    norm = jnp.sqrt(jnp.sum(wp * wp)) + 1e-16
    wgt = jnp.where(sel, jnp.tanh(z / norm), 0.0)

    acc = jnp.zeros((1, C), dtype=jnp.float32)
    for j in range(R):
        acc = acc + jax.lax.dot_general(
            wgt[j:j + 1, :], pred[j * M:(j + 1) * M, :],
            (((1,), (0,)), ((), ())), preferred_element_type=jnp.float32)
    xf_ref[:, :] = acc * (1.0 / K)


@functools.partial(jax.jit, static_argnames=())
def kernel(x, edge_index, batch, W, b, w_pool):
    del edge_index, batch
    b2 = b.reshape(1, C)
    wp2 = w_pool.reshape(1, C)
    x_final, pred = pl.pallas_call(
        _body,
        out_shape=(
            jax.ShapeDtypeStruct((1, C), jnp.float32),
            jax.ShapeDtypeStruct((N, C), jnp.float32),
        ),
        scratch_shapes=[pltpu.VMEM((R, M), jnp.float32)],
    )(x, W, b2, wp2)
    return (x_final, pred)


# speedup vs baseline: 6.9441x; 1.2034x over previous
import functools

import jax
import jax.numpy as jnp
from jax.experimental import pallas as pl
from jax.experimental.pallas import tpu as pltpu

N = 10000
D = 128
C = 16
K = 5000
R = 10
M = N // R


def _body(x_ref, w_ref, b_ref, wp_ref, xf_ref, pred_ref, zs_ref):
    x = x_ref[:, :]
    w = w_ref[:, :]
    b = b_ref[:, :]
    wp = wp_ref[:, :]

    pred = jax.lax.dot_general(
        x, w, (((1,), (1,)), ((), ())), preferred_element_type=jnp.float32
    ) + b
    pred_ref[:, :] = pred

    for j in range(R):
        zj = jax.lax.dot_general(
            wp, pred[j * M:(j + 1) * M, :], (((1,), (1,)), ((), ())),
            preferred_element_type=jnp.float32)
        zs_ref[j:j + 1, :] = zj
    z = zs_ref[:, :]

    u = jax.lax.bitcast_convert_type(z, jnp.uint32)
    sign = u >> jnp.uint32(31)
    flip = jnp.where(sign == jnp.uint32(1),
                     jnp.uint32(0xFFFFFFFF), jnp.uint32(0x80000000))
    key = u ^ flip

    def _cnt_ge(t):
        return jnp.sum((key >= t).astype(jnp.int32))

    kth = jnp.uint32(0)
    for sh in range(28, -1, -4):
        cnts = [_cnt_ge(kth | jnp.uint32(d << sh)) for d in range(1, 16)]
        digit = sum((c >= K).astype(jnp.uint32) for c in cnts)
        kth = kth | (digit << jnp.uint32(sh))

    above = key > kth
    m = jnp.sum(above.astype(jnp.int32))
    need = K - m

    tie = key == kth
    idx = (jax.lax.broadcasted_iota(jnp.int32, (R, M), 0) * M
           + jax.lax.broadcasted_iota(jnp.int32, (R, M), 1))

    def _cnt_lt(v):
        return jnp.sum((tie & (idx < v)).astype(jnp.int32))

    jstar = jnp.int32(0)
    for sh in range(12, -1, -4):
        cnts = [_cnt_lt(jstar | jnp.int32(d << sh)) for d in range(1, 16)]
        digit = sum((c < need).astype(jnp.int32) for c in cnts)
        jstar = jstar | (digit << sh)

    sel = above | (tie & (idx <= jstar))
    norm = jnp.sqrt(jnp.sum(wp * wp)) + 1e-16
    wgt = jnp.where(sel, jnp.tanh(z / norm), 0.0)

    acc = jnp.zeros((1, C), dtype=jnp.float32)
    for j in range(R):
        acc = acc + jax.lax.dot_general(
            wgt[j:j + 1, :], pred[j * M:(j + 1) * M, :],
            (((1,), (0,)), ((), ())), preferred_element_type=jnp.float32)
    xf_ref[:, :] = acc * (1.0 / K)


@functools.partial(jax.jit, static_argnames=())
def kernel(x, edge_index, batch, W, b, w_pool):
    del edge_index, batch
    b2 = b.reshape(1, C)
    wp2 = w_pool.reshape(1, C)
    x_final, pred = pl.pallas_call(
        _body,
        out_shape=(
            jax.ShapeDtypeStruct((1, C), jnp.float32),
            jax.ShapeDtypeStruct((N, C), jnp.float32),
        ),
        scratch_shapes=[pltpu.VMEM((R, M), jnp.float32)],
    )(x, W, b2, wp2)
    return (x_final, pred)
